# Initial kernel scaffold; baseline (speedup 1.0000x reference)
import jax
import jax.numpy as jnp
from jax.experimental import pallas as pl


def kernel(video, caption_ids, emb_table, txt_w, txt_b, vid_w, vid_b):
    raise NotImplementedError("write your pallas kernel here")


# speedup vs baseline: 1.0089x; 1.0089x over previous
import functools

import jax
import jax.numpy as jnp
from jax import lax
from jax.experimental import pallas as pl
from jax.experimental.pallas import tpu as pltpu
from jax.experimental.pallas import tpu_sc as plsc

B = 4096
L_SEQ = 200
D = 64
HALF = L_SEQ // 2

NC = 2
NS = 16
NW = NC * NS
RPW = B // NW


def _sc_pool_sums(emb_table, ids2):
    mesh = plsc.VectorSubcoreMesh(
        core_axis_name="c", subcore_axis_name="s", num_cores=NC, num_subcores=NS
    )

    @functools.partial(
        pl.kernel,
        mesh=mesh,
        compiler_params=pltpu.CompilerParams(use_tc_tiling_on_sc=False),
        out_type=jax.ShapeDtypeStruct((B, D), jnp.float32),
        scratch_types=[
            pltpu.VMEM((2 * RPW, HALF), jnp.int32),
            pltpu.VMEM((2, L_SEQ, D), jnp.float32),
            pltpu.VMEM((RPW, D), jnp.float32),
            pltpu.SemaphoreType.DMA,
            pltpu.SemaphoreType.DMA,
        ],
    )
    def k(emb_hbm, ids_hbm, out_hbm, ids_v, rows_v, out_v, sem0, sem1):
        wid = lax.axis_index("s") * NC + lax.axis_index("c")
        base2 = wid * (2 * RPW)
        pltpu.sync_copy(ids_hbm.at[pl.ds(base2, 2 * RPW)], ids_v)

        def descs(row, buf):
            sem = sem0 if buf == 0 else sem1
            d0 = pltpu.make_async_copy(
                emb_hbm.at[ids_v.at[2 * row]],
                rows_v.at[buf, pl.ds(0, HALF)],
                sem,
            )
            d1 = pltpu.make_async_copy(
                emb_hbm.at[ids_v.at[2 * row + 1]],
                rows_v.at[buf, pl.ds(HALF, HALF)],
                sem,
            )
            return d0, d1

        def start(row, buf):
            d0, d1 = descs(row, buf)
            d0.start()
            d1.start()

        def wait(row, buf):
            d0, d1 = descs(row, buf)
            d0.wait()
            d1.wait()

        def accum(row, buf):
            zero = jnp.zeros((16,), jnp.float32)

            def body(l, accs):
                return tuple(
                    accs[g] + rows_v[buf, l, pl.ds(g * 16, 16)] for g in range(4)
                )

            accs = lax.fori_loop(0, L_SEQ, body, (zero,) * 4)
            for g in range(4):
                out_v[row, pl.ds(g * 16, 16)] = accs[g]

        start(0, 0)
        start(1, 1)

        def pair(p, carry):
            i0 = 2 * p
            wait(i0, 0)
            accum(i0, 0)
            start(i0 + 2, 0)
            wait(i0 + 1, 1)
            accum(i0 + 1, 1)
            start(i0 + 3, 1)
            return carry

        lax.fori_loop(0, RPW // 2 - 1, pair, 0)
        wait(RPW - 2, 0)
        accum(RPW - 2, 0)
        wait(RPW - 1, 1)
        accum(RPW - 1, 1)
        pltpu.sync_copy(out_v, out_hbm.at[pl.ds(wid * RPW, RPW)])

    return k(emb_table, ids2)


def _tc_body(vid_ref, ids_ref, pooled_ref, wv_ref, vb_ref, wt_ref, tb_ref, out_ref):
    i = pl.program_id(0)
    v = (
        jnp.dot(vid_ref[...], wv_ref[...], preferred_element_type=jnp.float32)
        + vb_ref[...]
    )
    cnt = jnp.sum((ids_ref[...] != 0).astype(jnp.float32), axis=1, keepdims=True)
    x = pooled_ref[...] / jnp.maximum(cnt, 1.0)
    x = (
        jnp.dot(x, wt_ref[...], preferred_element_type=jnp.float32)
        + tb_ref[...]
    )
    vn = v / jnp.maximum(
        jnp.sqrt(jnp.sum(v * v, axis=1, keepdims=True)), 1e-12
    )
    xn = x / jnp.maximum(
        jnp.sqrt(jnp.sum(x * x, axis=1, keepdims=True)), 1e-12
    )
    cos = jnp.sum(vn * xn, axis=1) / jnp.maximum(
        jnp.sqrt(jnp.sum(vn * vn, axis=1)) * jnp.sqrt(jnp.sum(xn * xn, axis=1)),
        1e-8,
    )
    part = jnp.sum(1.0 - cos) * (1.0 / B)

    @pl.when(i == 0)
    def _():
        out_ref[...] = jnp.zeros_like(out_ref)

    out_ref[...] += jnp.reshape(part, (1, 1))


def _tc_finish(vid2, ids, pooled, w_vid, vid_b2, txt_wt, txt_b2):
    bs = 512
    grid = (B // bs,)
    out = pl.pallas_call(
        _tc_body,
        grid=grid,
        in_specs=[
            pl.BlockSpec((bs, 12 * 256), lambda i: (i, 0)),
            pl.BlockSpec((bs, L_SEQ), lambda i: (i, 0)),
            pl.BlockSpec((bs, D), lambda i: (i, 0)),
            pl.BlockSpec((12 * 256, D), lambda i: (0, 0)),
            pl.BlockSpec((1, D), lambda i: (0, 0)),
            pl.BlockSpec((D, D), lambda i: (0, 0)),
            pl.BlockSpec((1, D), lambda i: (0, 0)),
        ],
        out_specs=pl.BlockSpec((1, 1), lambda i: (0, 0)),
        out_shape=jax.ShapeDtypeStruct((1, 1), jnp.float32),
    )(vid2, ids, pooled, w_vid, vid_b2, txt_wt, txt_b2)
    return out[0, 0]


def kernel(video, caption_ids, emb_table, txt_w, txt_b, vid_w, vid_b):
    ids = caption_ids.astype(jnp.int32)
    pooled = _sc_pool_sums(emb_table, ids.reshape(B * 2, HALF))
    vid2 = video.reshape(B, 12 * 256)
    w_vid = jnp.tile(jnp.repeat(vid_w.T * (1.0 / 1024.0), 256, axis=0), (4, 1))
    loss = _tc_finish(
        vid2,
        ids,
        pooled,
        w_vid,
        vid_b.reshape(1, D),
        txt_w.T,
        txt_b.reshape(1, D),
    )
    return loss
